# baseline (speedup 1.0000x reference)
import functools

import numpy as np
import jax
import jax.numpy as jnp
from jax import lax
from jax.experimental import pallas as pl
from jax.experimental.pallas import tpu as pltpu
from jax.experimental.pallas import tpu_sc as plsc

VOCAB = 100000
D = 768
B = 4
S = 4096

NC = 2
NS = 16
NW = NC * NS
SW = S // NW
CS = 8
NCHUNK = SW // CS
NBUF = 3
LANES = 16
KSTEPS = D // LANES


def _trig_tables():
    w = 1.0 / 10000.0 ** (np.arange(0, D, 2, dtype=np.float64) / D)

    def cos_rows(v):
        return np.repeat(np.cos(v[:, None] * w), 2, axis=1)

    def sin_rows(v):
        s = np.sin(v[:, None] * w)
        out = np.zeros((len(v), D))
        out[:, 0::2], out[:, 1::2] = s, -s
        return out

    u = np.arange(NCHUNK, dtype=np.float64) * CS
    r = np.arange(CS, dtype=np.float64)
    s0 = np.arange(NW, dtype=np.float64) * SW
    c0, sn0 = np.cos(s0[:, None] * w), np.sin(s0[:, None] * w)
    A1 = np.zeros((NW, D))
    A2 = np.zeros((NW, D))
    A1[:, 0::2], A1[:, 1::2] = sn0, c0
    A2[:, 0::2], A2[:, 1::2] = c0, sn0
    stack = np.concatenate(
        [cos_rows(u), sin_rows(u), cos_rows(r), sin_rows(r), A1, A2], axis=0)
    return stack.astype(np.float32)


_TRIG = _trig_tables()


def _emb_body(x_hbm, table_hbm, trig_hbm, out_hbm,
              idx_v, rows_v, cu_v, su_v, cr_v, sr_v, a1_v, a2_v, p1_v, p2_v,
              g0, g1, g2, o0, o1, o2):
    gsems = (g0, g1, g2)
    osems = (o0, o1, o2)

    cid = lax.axis_index("c")
    sid = lax.axis_index("s")
    wid = sid * NC + cid
    s0 = wid * SW

    for bb in range(B):
        pltpu.sync_copy(x_hbm.at[bb, pl.ds(s0, SW)], idx_v.at[bb])
    pltpu.sync_copy(trig_hbm.at[pl.ds(0, NCHUNK)], cu_v)
    pltpu.sync_copy(trig_hbm.at[pl.ds(NCHUNK, NCHUNK)], su_v)
    pltpu.sync_copy(trig_hbm.at[pl.ds(2 * NCHUNK, CS)], cr_v)
    pltpu.sync_copy(trig_hbm.at[pl.ds(2 * NCHUNK + CS, CS)], sr_v)
    base_a = 2 * NCHUNK + 2 * CS
    pltpu.sync_copy(trig_hbm.at[base_a + wid], a1_v)
    pltpu.sync_copy(trig_hbm.at[base_a + NW + wid], a2_v)

    def start_gather(i):
        q = i % NBUF
        return tuple(
            pltpu.async_copy(table_hbm.at[idx_v.at[bb, pl.ds(i * CS, CS)]],
                             rows_v.at[q, bb], gsems[q])
            for bb in range(B))

    gathers = {0: start_gather(0)}
    outs = {}
    for i in range(NCHUNK):
        q = i % NBUF
        if i + 1 < NCHUNK:
            if i - 2 >= 0:
                for o in outs.pop(i - 2):
                    o.wait()
            gathers[i + 1] = start_gather(i + 1)

        @plsc.parallel_loop(0, KSTEPS, unroll=2)
        def _phase(k, _i=i):
            dk = pl.ds(pl.multiple_of(k * LANES, LANES), LANES)
            cu = cu_v[_i, dk]
            su = su_v[_i, dk]
            a1 = a1_v[dk]
            a2 = a2_v[dk]
            p1_v[dk] = a1 * cu + a2 * su
            p2_v[dk] = a2 * cu - a1 * su

        for g in gathers.pop(i):
            g.wait()

        @pl.loop(0, CS)
        def _sl(sl, _q=q):
            @plsc.parallel_loop(0, KSTEPS, unroll=2)
            def _add(k):
                dk = pl.ds(pl.multiple_of(k * LANES, LANES), LANES)
                pv = p1_v[dk] * cr_v[sl, dk] + p2_v[dk] * sr_v[sl, dk]
                for bb in range(B):
                    plsc.addupdate(rows_v.at[_q, bb, sl, dk], pv)

        outs[i] = tuple(
            pltpu.async_copy(rows_v.at[q, bb],
                             out_hbm.at[bb, pl.ds(s0 + i * CS, CS)], osems[q])
            for bb in range(B))
    for i in range(NCHUNK - NBUF, NCHUNK):
        for o in outs.pop(i):
            o.wait()


@functools.cache
def _emb():
    return pl.kernel(
        _emb_body,
        out_type=jax.ShapeDtypeStruct((B, S, D), jnp.float32),
        mesh=plsc.VectorSubcoreMesh(core_axis_name="c", subcore_axis_name="s",
                                    num_cores=NC, num_subcores=NS),
        scratch_types=[
            pltpu.VMEM((B, SW), jnp.int32),
            pltpu.VMEM((NBUF, B, CS, D), jnp.float32),
            pltpu.VMEM((NCHUNK, D), jnp.float32),
            pltpu.VMEM((NCHUNK, D), jnp.float32),
            pltpu.VMEM((CS, D), jnp.float32),
            pltpu.VMEM((CS, D), jnp.float32),
            pltpu.VMEM((D,), jnp.float32),
            pltpu.VMEM((D,), jnp.float32),
            pltpu.VMEM((D,), jnp.float32),
            pltpu.VMEM((D,), jnp.float32),
        ] + [pltpu.SemaphoreType.DMA] * (2 * NBUF),
    )


def kernel(x, table):
    return _emb()(x.astype(jnp.int32), table, jnp.asarray(_TRIG))


# speedup vs baseline: 1.0668x; 1.0668x over previous
import functools

import numpy as np
import jax
import jax.numpy as jnp
from jax import lax
from jax.experimental import pallas as pl
from jax.experimental.pallas import tpu as pltpu
from jax.experimental.pallas import tpu_sc as plsc

VOCAB = 100000
D = 768
B = 4
S = 4096

NC = 2
NS = 16
NW = NC * NS
SW = S // NW
CS = 8
NCHUNK = SW // CS
NBUF = 3
TBUF = 2
LANES = 16
KSTEPS = D // LANES


def _trig_tables():
    w = 1.0 / 10000.0 ** (np.arange(0, D, 2, dtype=np.float64) / D)
    t = np.arange(SW, dtype=np.float64)[:, None]
    c, s = np.cos(t * w), np.sin(t * w)
    C = np.repeat(c, 2, axis=1)
    Sg = np.zeros((SW, D))
    Sg[:, 0::2], Sg[:, 1::2] = s, -s
    s0 = (np.arange(NW, dtype=np.float64) * SW)[:, None]
    c0, sn0 = np.cos(s0 * w), np.sin(s0 * w)
    A1 = np.zeros((NW, D))
    A2 = np.zeros((NW, D))
    A1[:, 0::2], A1[:, 1::2] = sn0, c0
    A2[:, 0::2], A2[:, 1::2] = c0, sn0
    f32 = np.float32
    return C.astype(f32), Sg.astype(f32), A1.astype(f32), A2.astype(f32)


_TRIG = np.concatenate(_trig_tables(), axis=0)


def _emb_body(x_hbm, table_hbm, trig_hbm, out_hbm,
              idx_v, rows_v, c_v, sg_v, a1_v, a2_v, c_sh, sg_sh,
              g0, g1, g2, o0, o1, o2, t0, t1):
    gsems = (g0, g1, g2)
    osems = (o0, o1, o2)
    tsems = (t0, t1)

    cid = lax.axis_index("c")
    sid = lax.axis_index("s")
    wid = sid * NC + cid
    s0 = wid * SW

    @pl.when(sid == 0)
    def _stage():
        pltpu.sync_copy(trig_hbm.at[pl.ds(0, SW)], c_sh)
        pltpu.sync_copy(trig_hbm.at[pl.ds(SW, SW)], sg_sh)

    for bb in range(B):
        pltpu.sync_copy(x_hbm.at[bb, pl.ds(s0, SW)], idx_v.at[bb])
    pltpu.sync_copy(trig_hbm.at[2 * SW + wid], a1_v)
    pltpu.sync_copy(trig_hbm.at[2 * SW + NW + wid], a2_v)

    def start_gather(i):
        q = i % NBUF
        return tuple(
            pltpu.async_copy(table_hbm.at[idx_v.at[bb, pl.ds(i * CS, CS)]],
                             rows_v.at[q, bb], gsems[q])
            for bb in range(B))

    def start_trig(i):
        pt = i % TBUF
        a = pltpu.async_copy(c_sh.at[pl.ds(i * CS, CS)], c_v.at[pt], tsems[pt])
        b = pltpu.async_copy(sg_sh.at[pl.ds(i * CS, CS)], sg_v.at[pt],
                             tsems[pt])
        return a, b

    gathers = {0: start_gather(0)}
    plsc.subcore_barrier()
    trigs = {0: start_trig(0)}
    outs = {}
    for i in range(NCHUNK):
        q = i % NBUF
        pt = i % TBUF
        if i + 1 < NCHUNK:
            if i - 2 >= 0:
                for o in outs.pop(i - 2):
                    o.wait()
            gathers[i + 1] = start_gather(i + 1)
            trigs[i + 1] = start_trig(i + 1)
        for t in trigs.pop(i):
            t.wait()
        for g in gathers.pop(i):
            g.wait()

        @pl.loop(0, CS)
        def _sl(sl, _q=q, _pt=pt):
            @plsc.parallel_loop(0, KSTEPS, unroll=2)
            def _add(k):
                off = pl.multiple_of(k * LANES, LANES)
                dk = pl.ds(off, LANES)
                pv = (a1_v[dk] * c_v[_pt, sl, dk]
                      + a2_v[dk] * sg_v[_pt, sl, dk])
                for bb in range(B):
                    plsc.addupdate(rows_v.at[_q, bb, sl, dk], pv)

        outs[i] = tuple(
            pltpu.async_copy(rows_v.at[q, bb],
                             out_hbm.at[bb, pl.ds(s0 + i * CS, CS)], osems[q])
            for bb in range(B))
    for i in range(NCHUNK - NBUF, NCHUNK):
        for o in outs.pop(i):
            o.wait()


@functools.cache
def _emb():
    return pl.kernel(
        _emb_body,
        out_type=jax.ShapeDtypeStruct((B, S, D), jnp.float32),
        mesh=plsc.VectorSubcoreMesh(core_axis_name="c", subcore_axis_name="s",
                                    num_cores=NC, num_subcores=NS),
        scratch_types=[
            pltpu.VMEM((B, SW), jnp.int32),
            pltpu.VMEM((NBUF, B, CS, D), jnp.float32),
            pltpu.VMEM((TBUF, CS, D), jnp.float32),
            pltpu.VMEM((TBUF, CS, D), jnp.float32),
            pltpu.VMEM((D,), jnp.float32),
            pltpu.VMEM((D,), jnp.float32),
            pltpu.VMEM_SHARED((SW, D), jnp.float32),
            pltpu.VMEM_SHARED((SW, D), jnp.float32),
        ] + [pltpu.SemaphoreType.DMA] * (NBUF + NBUF + TBUF),
    )


def kernel(x, table):
    return _emb()(x.astype(jnp.int32), table, jnp.asarray(_TRIG))


# speedup vs baseline: 1.0988x; 1.0300x over previous
import functools

import numpy as np
import jax
import jax.numpy as jnp
from jax import lax
from jax.experimental import pallas as pl
from jax.experimental.pallas import tpu as pltpu
from jax.experimental.pallas import tpu_sc as plsc

VOCAB = 100000
D = 768
B = 4
S = 4096

NC = 2
NS = 16
NW = NC * NS
SW = S // NW
CS = 8
NCHUNK = SW // CS
NBUF = 3
TBUF = 2
LANES = 16
KSTEPS = D // LANES


def _trig_tables():
    w = 1.0 / 10000.0 ** (np.arange(0, D, 2, dtype=np.float64) / D)
    t = np.arange(SW, dtype=np.float64)[:, None]
    c, s = np.cos(t * w), np.sin(t * w)
    C = np.repeat(c, 2, axis=1)
    Sg = np.zeros((SW, D))
    Sg[:, 0::2], Sg[:, 1::2] = s, -s
    s0 = (np.arange(NW, dtype=np.float64) * SW)[:, None]
    c0, sn0 = np.cos(s0 * w), np.sin(s0 * w)
    A1 = np.zeros((NW, D))
    A2 = np.zeros((NW, D))
    A1[:, 0::2], A1[:, 1::2] = sn0, c0
    A2[:, 0::2], A2[:, 1::2] = c0, sn0
    f32 = np.float32
    return C.astype(f32), Sg.astype(f32), A1.astype(f32), A2.astype(f32)


_TRIG = np.concatenate(_trig_tables(), axis=0)


def _emb_body(x_hbm, table_hbm, trig_hbm, out_hbm,
              idx_v, rows_v, c_v, sg_v, a1_v, a2_v, c_sh, sg_sh,
              g0, g1, g2, o0, o1, o2, t0, t1):
    gsems = (g0, g1, g2)
    osems = (o0, o1, o2)
    tsems = (t0, t1)

    cid = lax.axis_index("c")
    sid = lax.axis_index("s")
    wid = sid * NC + cid
    s0 = wid * SW

    @pl.when(sid == 0)
    def _stage():
        pltpu.sync_copy(trig_hbm.at[pl.ds(0, SW)], c_sh)
        pltpu.sync_copy(trig_hbm.at[pl.ds(SW, SW)], sg_sh)

    for bb in range(B):
        pltpu.sync_copy(x_hbm.at[bb, pl.ds(s0, SW)], idx_v.at[bb])
    pltpu.sync_copy(trig_hbm.at[2 * SW + wid], a1_v)
    pltpu.sync_copy(trig_hbm.at[2 * SW + NW + wid], a2_v)

    def gather_descs(c, q):
        co = pl.multiple_of(c * CS, CS)
        return [(table_hbm.at[idx_v.at[bb, pl.ds(co, CS)]], rows_v.at[q, bb],
                 gsems[q]) for bb in range(B)]

    def trig_descs(c, pt):
        co = pl.multiple_of(c * CS, CS)
        return [(c_sh.at[pl.ds(co, CS)], c_v.at[pt], tsems[pt]),
                (sg_sh.at[pl.ds(co, CS)], sg_v.at[pt], tsems[pt])]

    def out_descs(c, q):
        return [(rows_v.at[q, bb], out_hbm.at[bb, pl.ds(s0 + c * CS, CS)],
                 osems[q]) for bb in range(B)]

    def start(descs):
        for d in descs:
            pltpu.async_copy(*d)

    def wait(descs):
        for d in descs:
            pltpu.make_async_copy(*d).wait()

    def compute(c, q, pt):
        @pl.loop(0, CS)
        def _sl(sl):
            @plsc.parallel_loop(0, KSTEPS, unroll=2)
            def _add(k):
                off = pl.multiple_of(k * LANES, LANES)
                dk = pl.ds(off, LANES)
                pv = (a1_v[dk] * c_v[pt, sl, dk]
                      + a2_v[dk] * sg_v[pt, sl, dk])
                for bb in range(B):
                    plsc.addupdate(rows_v.at[q, bb, sl, dk], pv)

    def chunk_step(c, j):
        q, pt = (2 + j) % NBUF, j % TBUF
        wait(out_descs(c - 2, j % NBUF))
        start(gather_descs(c + 1, j % NBUF))
        start(trig_descs(c + 1, (j + 1) % TBUF))
        wait(trig_descs(c, pt))
        wait(gather_descs(c, q))
        compute(c, q, pt)
        start(out_descs(c, q))

    start(gather_descs(0, 0))
    plsc.subcore_barrier()
    start(trig_descs(0, 0))
    for c in (0, 1):
        q, pt = c % NBUF, c % TBUF
        start(gather_descs(c + 1, (c + 1) % NBUF))
        start(trig_descs(c + 1, (c + 1) % TBUF))
        wait(trig_descs(c, pt))
        wait(gather_descs(c, q))
        compute(c, q, pt)
        start(out_descs(c, q))

    @pl.loop(0, NCHUNK - 4, step=2 * NBUF)
    def _rounds(i0):
        for j in range(2 * NBUF):
            chunk_step(i0 + 2 + j, j)

    for c in (NCHUNK - 2, NCHUNK - 1):
        q, pt = c % NBUF, c % TBUF
        wait(out_descs(c - 2, (c - 2) % NBUF))
        if c + 1 < NCHUNK:
            start(gather_descs(c + 1, (c + 1) % NBUF))
            start(trig_descs(c + 1, (c + 1) % TBUF))
        wait(trig_descs(c, pt))
        wait(gather_descs(c, q))
        compute(c, q, pt)
        start(out_descs(c, q))
    for c in (NCHUNK - 2, NCHUNK - 1):
        wait(out_descs(c, c % NBUF))


@functools.cache
def _emb():
    return pl.kernel(
        _emb_body,
        out_type=jax.ShapeDtypeStruct((B, S, D), jnp.float32),
        mesh=plsc.VectorSubcoreMesh(core_axis_name="c", subcore_axis_name="s",
                                    num_cores=NC, num_subcores=NS),
        scratch_types=[
            pltpu.VMEM((B, SW), jnp.int32),
            pltpu.VMEM((NBUF, B, CS, D), jnp.float32),
            pltpu.VMEM((TBUF, CS, D), jnp.float32),
            pltpu.VMEM((TBUF, CS, D), jnp.float32),
            pltpu.VMEM((D,), jnp.float32),
            pltpu.VMEM((D,), jnp.float32),
            pltpu.VMEM_SHARED((SW, D), jnp.float32),
            pltpu.VMEM_SHARED((SW, D), jnp.float32),
        ] + [pltpu.SemaphoreType.DMA] * (NBUF + NBUF + TBUF),
    )


def kernel(x, table):
    return _emb()(x.astype(jnp.int32), table, jnp.asarray(_TRIG))
